# baseline (speedup 1.0000x reference)
import functools

import jax
import jax.numpy as jnp
from jax import lax
from jax.experimental import pallas as pl
from jax.experimental.pallas import tpu as pltpu
from jax.experimental.pallas import tpu_sc as plsc

_ROWS = 262144
_COLS = 128
_N = _ROWS * _COLS
_NC, _NS, _L = 2, 16, 16
_NW = _NC * _NS
_PER_W = _N // _NW
_CHUNK = 16384
_NCHUNK = _PER_W // _CHUNK


def _sc_body(a_hbm, b_hbm, o_hbm, abuf, bbuf, obuf, sem_a, sem_b):
    c = lax.axis_index("c")
    s = lax.axis_index("s")
    wid = s * _NC + c
    base = wid * _PER_W

    def chunk_body(g, carry):
        off = base + g * _CHUNK
        ca = pltpu.make_async_copy(a_hbm.at[pl.ds(off, _CHUNK)], abuf, sem_a)
        cb = pltpu.make_async_copy(b_hbm.at[pl.ds(off, _CHUNK)], bbuf, sem_b)
        ca.start()
        cb.start()
        ca.wait()
        cb.wait()

        def mul_body(i, c2):
            sl = pl.ds(i * _L, _L)
            obuf[sl] = abuf[sl] * bbuf[sl]
            return c2

        lax.fori_loop(0, _CHUNK // _L, mul_body, 0, unroll=8)
        pltpu.sync_copy(obuf, o_hbm.at[pl.ds(off, _CHUNK)])
        return carry

    lax.fori_loop(0, _NCHUNK, chunk_body, 0)

    @pl.when(wid == 0)
    def _():
        pltpu.sync_copy(b_hbm.at[pl.ds(0, _COLS)], abuf.at[pl.ds(0, _COLS)])
        for j in range(_COLS // _L):
            sl = pl.ds(j * _L, _L)
            obuf[sl] = 2.0 * abuf[sl]
        pltpu.sync_copy(obuf.at[pl.ds(0, _COLS)], o_hbm.at[pl.ds(0, _COLS)])


@jax.jit
def kernel(a, b):
    mesh = plsc.VectorSubcoreMesh(core_axis_name="c", subcore_axis_name="s")
    run = functools.partial(
        pl.kernel,
        mesh=mesh,
        out_type=jax.ShapeDtypeStruct((_N,), jnp.float32),
        scratch_types=[
            pltpu.VMEM((_CHUNK,), jnp.float32),
            pltpu.VMEM((_CHUNK,), jnp.float32),
            pltpu.VMEM((_CHUNK,), jnp.float32),
            pltpu.SemaphoreType.DMA,
            pltpu.SemaphoreType.DMA,
        ],
    )(_sc_body)
    out = run(a.reshape(_N), b.reshape(_N))
    return out.reshape(_ROWS, _COLS)


# speedup vs baseline: 1.4010x; 1.4010x over previous
import functools

import jax
import jax.numpy as jnp
from jax import lax
from jax.experimental import pallas as pl
from jax.experimental.pallas import tpu as pltpu
from jax.experimental.pallas import tpu_sc as plsc

_ROWS = 262144
_COLS = 128
_N = _ROWS * _COLS
_NC, _NS, _L = 2, 16, 16
_NW = _NC * _NS
_PER_W = _N // _NW
_CHUNK = 16384
_NCHUNK = _PER_W // _CHUNK


def _sc_body(a_hbm, b_hbm, o_hbm,
             abuf0, abuf1, bbuf0, bbuf1, obuf0, obuf1,
             sa0, sa1, sb0, sb1, so0, so1):
    c = lax.axis_index("c")
    s = lax.axis_index("s")
    wid = s * _NC + c
    base = wid * _PER_W

    abufs, bbufs, obufs = (abuf0, abuf1), (bbuf0, bbuf1), (obuf0, obuf1)
    sas, sbs, sos = (sa0, sa1), (sb0, sb1), (so0, so1)

    def in_copies(g, sl):
        off = base + g * _CHUNK
        ca = pltpu.make_async_copy(a_hbm.at[pl.ds(off, _CHUNK)], abufs[sl], sas[sl])
        cb = pltpu.make_async_copy(b_hbm.at[pl.ds(off, _CHUNK)], bbufs[sl], sbs[sl])
        return ca, cb

    def out_copy(g, sl):
        off = base + g * _CHUNK
        return pltpu.make_async_copy(obufs[sl], o_hbm.at[pl.ds(off, _CHUNK)], sos[sl])

    for g in (0, 1):
        ca, cb = in_copies(g, g)
        ca.start()
        cb.start()

    for g in range(_NCHUNK):
        sl = g % 2
        if g >= 2:
            out_copy(g - 2, sl).wait()
        ca, cb = in_copies(g, sl)
        ca.wait()
        cb.wait()

        def mul_body(i, c2, sl=sl):
            v = pl.ds(i * _L, _L)
            obufs[sl][v] = abufs[sl][v] * bbufs[sl][v]
            return c2

        lax.fori_loop(0, _CHUNK // _L, mul_body, 0, unroll=8)
        out_copy(g, sl).start()
        if g + 2 < _NCHUNK:
            ca, cb = in_copies(g + 2, sl)
            ca.start()
            cb.start()

    out_copy(_NCHUNK - 2, 0).wait()
    out_copy(_NCHUNK - 1, 1).wait()

    @pl.when(wid == 0)
    def _():
        pltpu.sync_copy(b_hbm.at[pl.ds(0, _COLS)], abuf0.at[pl.ds(0, _COLS)])
        for j in range(_COLS // _L):
            v = pl.ds(j * _L, _L)
            obuf0[v] = 2.0 * abuf0[v]
        pltpu.sync_copy(obuf0.at[pl.ds(0, _COLS)], o_hbm.at[pl.ds(0, _COLS)])


@jax.jit
def kernel(a, b):
    mesh = plsc.VectorSubcoreMesh(core_axis_name="c", subcore_axis_name="s")
    run = functools.partial(
        pl.kernel,
        mesh=mesh,
        out_type=jax.ShapeDtypeStruct((_N,), jnp.float32),
        scratch_types=[
            pltpu.VMEM((_CHUNK,), jnp.float32),
            pltpu.VMEM((_CHUNK,), jnp.float32),
            pltpu.VMEM((_CHUNK,), jnp.float32),
            pltpu.VMEM((_CHUNK,), jnp.float32),
            pltpu.VMEM((_CHUNK,), jnp.float32),
            pltpu.VMEM((_CHUNK,), jnp.float32),
            pltpu.SemaphoreType.DMA,
            pltpu.SemaphoreType.DMA,
            pltpu.SemaphoreType.DMA,
            pltpu.SemaphoreType.DMA,
            pltpu.SemaphoreType.DMA,
            pltpu.SemaphoreType.DMA,
        ],
    )(_sc_body)
    out = run(a.reshape(_N), b.reshape(_N))
    return out.reshape(_ROWS, _COLS)
